# Initial kernel scaffold; baseline (speedup 1.0000x reference)
import jax
import jax.numpy as jnp
from jax.experimental import pallas as pl


def kernel(x, edge_index, edge_attr, w1a, b1a, w2a, w3a, b3a, w1b, b1b, w2b, w3b, b3b, we1, be1, we2, be2):
    raise NotImplementedError("write your pallas kernel here")


# speedup vs baseline: 4.5074x; 4.5074x over previous
import functools

import jax
import jax.numpy as jnp
from jax import lax
from jax.experimental import pallas as pl
from jax.experimental.pallas import tpu as pltpu
from jax.experimental.pallas import tpu_sc as plsc

N_NODES = 50000
N_EDGES = 800000
CH = 64
LANES = 16
NUM_SC = 2
NUM_SUBCORES = 16
NUM_TILES = NUM_SC * NUM_SUBCORES
CHUNK = 1024
STREAM = 128
NSTREAM = CHUNK // STREAM
NCHUNK = 25
PER_TILE = NCHUNK * CHUNK
EDGES_PAD = NUM_TILES * PER_TILE
NODES_PAD = 50048
ROWS_PER_SUBCORE = NODES_PAD // NUM_SUBCORES

_F32 = jnp.float32


def _sc_mesh():
    return plsc.VectorSubcoreMesh(core_axis_name="c", subcore_axis_name="s")


_SC_PARAMS = pltpu.CompilerParams(use_tc_tiling_on_sc=False)


def _sc_scatter_body(table, row2, col2, ew16, zrows, out,
                     rowv, colv, ewv, msgv, acc, sem):
    c = lax.axis_index("c")
    s = lax.axis_index("s")
    tile = s * NUM_SC + c

    srow = pl.multiple_of(s * ROWS_PER_SUBCORE, 8)
    pltpu.sync_copy(zrows, acc.at[pl.ds(srow, ROWS_PER_SUBCORE)])
    plsc.subcore_barrier()

    def chunk_body(i, carry):
        base = pl.multiple_of(tile * PER_TILE + i * CHUNK, CHUNK)
        r0 = pl.multiple_of(base // STREAM, NSTREAM)
        pltpu.sync_copy(row2.at[pl.ds(r0, NSTREAM)], rowv)
        pltpu.sync_copy(col2.at[pl.ds(r0, NSTREAM)], colv)
        pltpu.sync_copy(ew16.at[pl.ds(base, CHUNK)], ewv)
        cps = [pltpu.async_copy(table.at[rowv.at[j]],
                                msgv.at[pl.ds(j * STREAM, STREAM)], sem)
               for j in range(NSTREAM)]
        for cp in cps:
            cp.wait()

        def mul_body(rr, carry2):
            rb = rr * 8
            for u in range(8):
                msgv[rb + u, :] = msgv[rb + u, :] * ewv[rb + u, :]
            return carry2

        lax.fori_loop(0, CHUNK // 8, mul_body, 0)
        for j in range(NSTREAM):
            pltpu.sync_copy(msgv.at[pl.ds(j * STREAM, STREAM)],
                            acc.at[colv.at[j]], add=True)
        return carry

    lax.fori_loop(0, NCHUNK, chunk_body, 0)

    plsc.subcore_barrier()
    pltpu.sync_copy(acc.at[pl.ds(srow, ROWS_PER_SUBCORE)],
                    out.at[c, pl.ds(srow, ROWS_PER_SUBCORE)])


def _sc_scatter(table, row2, col2, ew16, zrows):
    f = pl.kernel(
        _sc_scatter_body,
        out_type=jax.ShapeDtypeStruct((NUM_SC, NODES_PAD, LANES), _F32),
        mesh=_sc_mesh(),
        scratch_types=[
            pltpu.VMEM((NSTREAM, STREAM), jnp.int32),
            pltpu.VMEM((NSTREAM, STREAM), jnp.int32),
            pltpu.VMEM((CHUNK, LANES), _F32),
            pltpu.VMEM((CHUNK, LANES), _F32),
            pltpu.VMEM_SHARED((NODES_PAD, LANES), _F32),
            pltpu.SemaphoreType.DMA,
        ],
        compiler_params=_SC_PARAMS,
    )
    return f(table, row2, col2, ew16, zrows)


def _sc_gather_body(q2, row2, col2, efr, efc, idxv, buf, sem):
    c = lax.axis_index("c")
    s = lax.axis_index("s")
    tile = s * NUM_SC + c

    def one_side(idx2, dst, base, r0):
        pltpu.sync_copy(idx2.at[pl.ds(r0, NSTREAM)], idxv)
        cps = [pltpu.async_copy(q2.at[idxv.at[j]],
                                buf.at[pl.ds(j * STREAM, STREAM)], sem)
               for j in range(NSTREAM)]
        for cp in cps:
            cp.wait()
        pltpu.sync_copy(buf, dst.at[pl.ds(base, CHUNK)])

    def chunk_body(i, carry):
        base = pl.multiple_of(tile * PER_TILE + i * CHUNK, CHUNK)
        r0 = pl.multiple_of(base // STREAM, NSTREAM)
        one_side(row2, efr, base, r0)
        one_side(col2, efc, base, r0)
        return carry

    lax.fori_loop(0, NCHUNK, chunk_body, 0)


def _sc_gather(q2, row2, col2):
    f = pl.kernel(
        _sc_gather_body,
        out_type=[jax.ShapeDtypeStruct((EDGES_PAD, CH), _F32),
                  jax.ShapeDtypeStruct((EDGES_PAD, CH), _F32)],
        mesh=_sc_mesh(),
        scratch_types=[
            pltpu.VMEM((NSTREAM, STREAM), jnp.int32),
            pltpu.VMEM((CHUNK, CH), _F32),
            pltpu.SemaphoreType.DMA,
        ],
        compiler_params=_SC_PARAMS,
    )
    return f(q2, row2, col2)


def _g1_body(p1_ref, x2_ref, x3_ref, g1_ref):
    a = p1_ref[0] + p1_ref[1]
    g1_ref[...] = a + x2_ref[...] + a[:, 12:13] * x3_ref[...]


def _build_g1(part1, xpad2, xpad3):
    r = ROWS_PER_SUBCORE
    return pl.pallas_call(
        _g1_body,
        grid=(NODES_PAD // r,),
        in_specs=[
            pl.BlockSpec((NUM_SC, r, LANES), lambda i: (0, i, 0)),
            pl.BlockSpec((r, LANES), lambda i: (i, 0)),
            pl.BlockSpec((r, LANES), lambda i: (i, 0)),
        ],
        out_specs=pl.BlockSpec((r, LANES), lambda i: (i, 0)),
        out_shape=jax.ShapeDtypeStruct((NODES_PAD, LANES), _F32),
    )(part1, xpad2, xpad3)


def _q2_body(p2_ref, g1_ref, wg_ref, wh_ref, wk_ref, q2_ref):
    g = p2_ref[0] + p2_ref[1]
    g1 = g1_ref[...]
    q2_ref[...] = (
        jnp.dot(g, wg_ref[...], preferred_element_type=_F32)
        + jnp.dot(g1, wh_ref[...], preferred_element_type=_F32)
        + jnp.dot(g1 * g1[:, 12:13], wk_ref[...], preferred_element_type=_F32))


def _compute_q2(part2, g1, wg, wh, wk):
    r = ROWS_PER_SUBCORE
    wspec = pl.BlockSpec((LANES, CH), lambda i: (0, 0))
    return pl.pallas_call(
        _q2_body,
        grid=(NODES_PAD // r,),
        in_specs=[
            pl.BlockSpec((NUM_SC, r, LANES), lambda i: (0, i, 0)),
            pl.BlockSpec((r, LANES), lambda i: (i, 0)),
            wspec, wspec, wspec,
        ],
        out_specs=pl.BlockSpec((r, CH), lambda i: (i, 0)),
        out_shape=jax.ShapeDtypeStruct((NODES_PAD, CH), _F32),
    )(part2, g1, wg, wh, wk)


def _head_body(efr_ref, efc_ref, w1r_ref, w1c_ref, be1_ref, w2_ref, be2_ref,
               out_ref):
    h = jnp.dot(efr_ref[...], w1r_ref[...], preferred_element_type=_F32)
    h = h + jnp.dot(efc_ref[...], w1c_ref[...], preferred_element_type=_F32)
    h = jnp.maximum(h + be1_ref[...], 0.0)
    out_ref[...] = jnp.dot(h, w2_ref[...], preferred_element_type=_F32) + be2_ref[...]


def _head(efr, efc, w1r, w1c, be1, w2, be2):
    b = 1600
    return pl.pallas_call(
        _head_body,
        grid=(N_EDGES // b,),
        in_specs=[
            pl.BlockSpec((b, CH), lambda i: (i, 0)),
            pl.BlockSpec((b, CH), lambda i: (i, 0)),
            pl.BlockSpec((CH, 4 * CH), lambda i: (0, 0)),
            pl.BlockSpec((CH, 4 * CH), lambda i: (0, 0)),
            pl.BlockSpec((1, 4 * CH), lambda i: (0, 0)),
            pl.BlockSpec((4 * CH, 4), lambda i: (0, 0)),
            pl.BlockSpec((1, 4), lambda i: (0, 0)),
        ],
        out_specs=pl.BlockSpec((b, 4), lambda i: (i, 0)),
        out_shape=jax.ShapeDtypeStruct((N_EDGES, 4), _F32),
    )(efr, efc, w1r, w1c, be1, w2, be2)


def kernel(x, edge_index, edge_attr, w1a, b1a, w2a, w3a, b3a,
           w1b, b1b, w2b, w3b, b3b, we1, be1, we2, be2):
    row = edge_index[0]
    col = edge_index[1]
    ew = edge_attr[:, 0]

    pad = EDGES_PAD - N_EDGES
    pad_idx = (jnp.arange(pad, dtype=jnp.int32) * 97) % N_NODES
    rowp = jnp.concatenate([row, pad_idx]).reshape(-1, STREAM)
    colp = jnp.concatenate([col, pad_idx]).reshape(-1, STREAM)
    ewp = jnp.concatenate([ew, jnp.zeros((pad,), _F32)])
    ew16 = jnp.broadcast_to(ewp[:, None], (EDGES_PAD, LANES))

    xp = jnp.pad(x, ((0, NODES_PAD - N_NODES), (0, 0)))
    z4 = jnp.zeros((NODES_PAD, 4), _F32)
    one1 = jnp.ones((NODES_PAD, 1), _F32)
    xpad = jnp.concatenate(
        [xp, jnp.zeros((NODES_PAD, 8), _F32), one1, jnp.zeros((NODES_PAD, 3), _F32)], axis=1)
    xpad2 = jnp.concatenate(
        [z4, xp, jnp.zeros((NODES_PAD, 5), _F32), one1, jnp.zeros((NODES_PAD, 2), _F32)], axis=1)
    xpad3 = jnp.concatenate([jnp.zeros((NODES_PAD, 8), _F32), xp, z4], axis=1)
    zrows = jnp.zeros((ROWS_PER_SUBCORE, LANES), _F32)

    part1 = _sc_scatter(xpad, rowp, colp, ew16, zrows)
    g1 = _build_g1(part1, xpad2, xpad3)
    part2 = _sc_scatter(g1, rowp, colp, ew16, zrows)

    wq = jnp.concatenate(
        [w1a, w3a, -w2a, b1a[None], b3a[None], jnp.zeros((2, CH), _F32)], axis=0)
    wg = wq @ w1b
    wh = (wq @ w3b).at[12].add(b1b).at[13].add(b3b)
    wk = -(wq @ w2b)

    q2 = _compute_q2(part2, g1, wg, wh, wk)
    efr, efc = _sc_gather(q2, rowp, colp)
    return _head(efr, efc, we1[:CH], we1[CH:], be1[None], we2, be2[None])


# speedup vs baseline: 4.9902x; 1.1071x over previous
import functools

import jax
import jax.numpy as jnp
from jax import lax
from jax.experimental import pallas as pl
from jax.experimental.pallas import tpu as pltpu
from jax.experimental.pallas import tpu_sc as plsc

N_NODES = 50000
N_EDGES = 800000
CH = 64
LANES = 16
NUM_SC = 2
NUM_SUBCORES = 16
NUM_TILES = NUM_SC * NUM_SUBCORES
CHUNK = 1024
STREAM = 128
NSTREAM = CHUNK // STREAM
NCHUNK = 25
PER_TILE = NCHUNK * CHUNK
EDGES_PAD = NUM_TILES * PER_TILE
NODES_PAD = 50048
ROWS_PER_SUBCORE = NODES_PAD // NUM_SUBCORES

_F32 = jnp.float32


def _sc_mesh():
    return plsc.VectorSubcoreMesh(core_axis_name="c", subcore_axis_name="s")


_SC_PARAMS = pltpu.CompilerParams(use_tc_tiling_on_sc=False)


def _sc_scatter_body(table, row2, col2, ew16, zrows, out,
                     rowv, colv, ewv, msgv, acc, sem):
    c = lax.axis_index("c")
    s = lax.axis_index("s")
    tile = s * NUM_SC + c

    srow = pl.multiple_of(s * ROWS_PER_SUBCORE, 8)
    pltpu.sync_copy(zrows, acc.at[pl.ds(srow, ROWS_PER_SUBCORE)])
    plsc.subcore_barrier()

    def chunk_body(i, carry):
        base = pl.multiple_of(tile * PER_TILE + i * CHUNK, CHUNK)
        r0 = pl.multiple_of(base // STREAM, NSTREAM)
        pltpu.sync_copy(row2.at[pl.ds(r0, NSTREAM)], rowv)
        pltpu.sync_copy(col2.at[pl.ds(r0, NSTREAM)], colv)
        pltpu.sync_copy(ew16.at[pl.ds(base, CHUNK)], ewv)
        cps = [pltpu.async_copy(table.at[rowv.at[j]],
                                msgv.at[pl.ds(j * STREAM, STREAM)], sem)
               for j in range(NSTREAM)]
        for cp in cps:
            cp.wait()

        def mul_body(rr, carry2):
            rb = rr * 8
            for u in range(8):
                msgv[rb + u, :] = msgv[rb + u, :] * ewv[rb + u, :]
            return carry2

        lax.fori_loop(0, CHUNK // 8, mul_body, 0)
        for j in range(NSTREAM):
            pltpu.sync_copy(msgv.at[pl.ds(j * STREAM, STREAM)],
                            acc.at[colv.at[j]], add=True)
        return carry

    lax.fori_loop(0, NCHUNK, chunk_body, 0)

    plsc.subcore_barrier()
    pltpu.sync_copy(acc.at[pl.ds(srow, ROWS_PER_SUBCORE)],
                    out.at[c, pl.ds(srow, ROWS_PER_SUBCORE)])


def _sc_scatter(table, row2, col2, ew16, zrows):
    f = pl.kernel(
        _sc_scatter_body,
        out_type=jax.ShapeDtypeStruct((NUM_SC, NODES_PAD, LANES), _F32),
        mesh=_sc_mesh(),
        scratch_types=[
            pltpu.VMEM((NSTREAM, STREAM), jnp.int32),
            pltpu.VMEM((NSTREAM, STREAM), jnp.int32),
            pltpu.VMEM((CHUNK, LANES), _F32),
            pltpu.VMEM((CHUNK, LANES), _F32),
            pltpu.VMEM_SHARED((NODES_PAD, LANES), _F32),
            pltpu.SemaphoreType.DMA,
        ],
        compiler_params=_SC_PARAMS,
    )
    return f(table, row2, col2, ew16, zrows)


def _sc_gather_body(q2, row2, col2, efr, efc, idxv, buf, sem):
    c = lax.axis_index("c")
    s = lax.axis_index("s")
    tile = s * NUM_SC + c

    def one_side(idx2, dst, base, r0):
        pltpu.sync_copy(idx2.at[pl.ds(r0, NSTREAM)], idxv)
        cps = [pltpu.async_copy(q2.at[idxv.at[j]],
                                buf.at[pl.ds(j * STREAM, STREAM)], sem)
               for j in range(NSTREAM)]
        for cp in cps:
            cp.wait()
        pltpu.sync_copy(buf, dst.at[pl.ds(base, CHUNK)])

    def chunk_body(i, carry):
        base = pl.multiple_of(tile * PER_TILE + i * CHUNK, CHUNK)
        r0 = pl.multiple_of(base // STREAM, NSTREAM)
        one_side(row2, efr, base, r0)
        one_side(col2, efc, base, r0)
        return carry

    lax.fori_loop(0, NCHUNK, chunk_body, 0)


def _sc_gather(q2, row2, col2):
    f = pl.kernel(
        _sc_gather_body,
        out_type=[jax.ShapeDtypeStruct((EDGES_PAD, CH), _F32),
                  jax.ShapeDtypeStruct((EDGES_PAD, CH), _F32)],
        mesh=_sc_mesh(),
        scratch_types=[
            pltpu.VMEM((NSTREAM, STREAM), jnp.int32),
            pltpu.VMEM((CHUNK, CH), _F32),
            pltpu.SemaphoreType.DMA,
        ],
        compiler_params=_SC_PARAMS,
    )
    return f(q2, row2, col2)


def _g1_body(p1_ref, x2_ref, x3_ref, g1_ref):
    a = p1_ref[0] + p1_ref[1]
    g1_ref[...] = a + x2_ref[...] + a[:, 12:13] * x3_ref[...]


def _build_g1(part1, xpad2, xpad3):
    r = ROWS_PER_SUBCORE
    return pl.pallas_call(
        _g1_body,
        grid=(NODES_PAD // r,),
        in_specs=[
            pl.BlockSpec((NUM_SC, r, LANES), lambda i: (0, i, 0)),
            pl.BlockSpec((r, LANES), lambda i: (i, 0)),
            pl.BlockSpec((r, LANES), lambda i: (i, 0)),
        ],
        out_specs=pl.BlockSpec((r, LANES), lambda i: (i, 0)),
        out_shape=jax.ShapeDtypeStruct((NODES_PAD, LANES), _F32),
    )(part1, xpad2, xpad3)


def _q2_body(p2_ref, g1_ref, wg_ref, wh_ref, wk_ref, q2_ref):
    g = p2_ref[0] + p2_ref[1]
    g1 = g1_ref[...]
    q2_ref[...] = (
        jnp.dot(g, wg_ref[...], preferred_element_type=_F32)
        + jnp.dot(g1, wh_ref[...], preferred_element_type=_F32)
        + jnp.dot(g1 * g1[:, 12:13], wk_ref[...], preferred_element_type=_F32))


def _compute_q2(part2, g1, wg, wh, wk):
    r = ROWS_PER_SUBCORE
    wspec = pl.BlockSpec((LANES, CH), lambda i: (0, 0))
    return pl.pallas_call(
        _q2_body,
        grid=(NODES_PAD // r,),
        in_specs=[
            pl.BlockSpec((NUM_SC, r, LANES), lambda i: (0, i, 0)),
            pl.BlockSpec((r, LANES), lambda i: (i, 0)),
            wspec, wspec, wspec,
        ],
        out_specs=pl.BlockSpec((r, CH), lambda i: (i, 0)),
        out_shape=jax.ShapeDtypeStruct((NODES_PAD, CH), _F32),
    )(part2, g1, wg, wh, wk)


def _head_body(efr_ref, efc_ref, w1_ref, be1_ref, w2_ref, be2_ref, out_ref):
    ef = jnp.concatenate([efr_ref[...], efc_ref[...]], axis=1)
    h = jnp.dot(ef, w1_ref[...], preferred_element_type=_F32)
    h = jnp.maximum(h + be1_ref[...], 0.0)
    out_ref[...] = jnp.dot(h, w2_ref[...], preferred_element_type=_F32) + be2_ref[...]


def _head(efr, efc, w1, be1, w2, be2, n_edges):
    b = 8000
    return pl.pallas_call(
        _head_body,
        grid=(n_edges // b,),
        in_specs=[
            pl.BlockSpec((b, CH), lambda i: (i, 0)),
            pl.BlockSpec((b, CH), lambda i: (i, 0)),
            pl.BlockSpec((2 * CH, 4 * CH), lambda i: (0, 0)),
            pl.BlockSpec((1, 4 * CH), lambda i: (0, 0)),
            pl.BlockSpec((4 * CH, 4), lambda i: (0, 0)),
            pl.BlockSpec((1, 4), lambda i: (0, 0)),
        ],
        out_specs=pl.BlockSpec((b, 4), lambda i: (i, 0)),
        out_shape=jax.ShapeDtypeStruct((n_edges, 4), _F32),
    )(efr, efc, w1, be1, w2, be2)


def kernel(x, edge_index, edge_attr, w1a, b1a, w2a, w3a, b3a,
           w1b, b1b, w2b, w3b, b3b, we1, be1, we2, be2):
    row = edge_index[0]
    col = edge_index[1]
    ew = edge_attr[:, 0]

    pad = EDGES_PAD - N_EDGES
    pad_idx = (jnp.arange(pad, dtype=jnp.int32) * 97) % N_NODES
    rowp = jnp.concatenate([row, pad_idx]).reshape(-1, STREAM)
    colp = jnp.concatenate([col, pad_idx]).reshape(-1, STREAM)
    ewp = jnp.concatenate([ew, jnp.zeros((pad,), _F32)])
    ew16 = jnp.broadcast_to(ewp[:, None], (EDGES_PAD, LANES))

    xp = jnp.pad(x, ((0, NODES_PAD - N_NODES), (0, 0)))
    z4 = jnp.zeros((NODES_PAD, 4), _F32)
    one1 = jnp.ones((NODES_PAD, 1), _F32)
    xpad = jnp.concatenate(
        [xp, jnp.zeros((NODES_PAD, 8), _F32), one1, jnp.zeros((NODES_PAD, 3), _F32)], axis=1)
    xpad2 = jnp.concatenate(
        [z4, xp, jnp.zeros((NODES_PAD, 5), _F32), one1, jnp.zeros((NODES_PAD, 2), _F32)], axis=1)
    xpad3 = jnp.concatenate([jnp.zeros((NODES_PAD, 8), _F32), xp, z4], axis=1)
    zrows = jnp.zeros((ROWS_PER_SUBCORE, LANES), _F32)

    part1 = _sc_scatter(xpad, rowp, colp, ew16, zrows)
    g1 = _build_g1(part1, xpad2, xpad3)
    part2 = _sc_scatter(g1, rowp, colp, ew16, zrows)

    wq = jnp.concatenate(
        [w1a, w3a, -w2a, b1a[None], b3a[None], jnp.zeros((2, CH), _F32)], axis=0)
    wg = wq @ w1b
    wh = (wq @ w3b).at[12].add(b1b).at[13].add(b3b)
    wk = -(wq @ w2b)

    q2 = _compute_q2(part2, g1, wg, wh, wk)
    efr, efc = _sc_gather(q2, rowp, colp)
    return _head(efr, efc, we1, be1[None], we2, be2[None], N_EDGES)


# speedup vs baseline: 6.7216x; 1.3470x over previous
import functools

import jax
import jax.numpy as jnp
from jax import lax
from jax.experimental import pallas as pl
from jax.experimental.pallas import tpu as pltpu
from jax.experimental.pallas import tpu_sc as plsc

N_NODES = 50000
N_EDGES = 800000
CH = 64
LANES = 16
NUM_SC = 2
NUM_SUBCORES = 16
NUM_TILES = NUM_SC * NUM_SUBCORES
CHUNK = 1024
STREAM = 128
NSTREAM = CHUNK // STREAM
NCHUNK = 25
PER_TILE = NCHUNK * CHUNK
EDGES_PAD = NUM_TILES * PER_TILE
NODES_PAD = 50048
ROWS_PER_SUBCORE = NODES_PAD // NUM_SUBCORES

_F32 = jnp.float32


def _sc_mesh():
    return plsc.VectorSubcoreMesh(core_axis_name="c", subcore_axis_name="s")


_SC_PARAMS = pltpu.CompilerParams(use_tc_tiling_on_sc=False)


def _sc_scatter_body(table, row2, col2, ew16, zrows, out,
                     rowv, colv, ewv, msgv, acc, sem):
    c = lax.axis_index("c")
    s = lax.axis_index("s")
    tile = s * NUM_SC + c

    srow = pl.multiple_of(s * ROWS_PER_SUBCORE, 8)
    pltpu.sync_copy(zrows, acc.at[pl.ds(srow, ROWS_PER_SUBCORE)])
    plsc.subcore_barrier()

    def chunk_body(i, carry):
        base = pl.multiple_of(tile * PER_TILE + i * CHUNK, CHUNK)
        r0 = pl.multiple_of(base // STREAM, NSTREAM)
        pltpu.sync_copy(row2.at[pl.ds(r0, NSTREAM)], rowv)
        pltpu.sync_copy(col2.at[pl.ds(r0, NSTREAM)], colv)
        pltpu.sync_copy(ew16.at[pl.ds(base, CHUNK)], ewv)
        cps = [pltpu.async_copy(table.at[rowv.at[j]],
                                msgv.at[pl.ds(j * STREAM, STREAM)], sem)
               for j in range(NSTREAM)]
        for cp in cps:
            cp.wait()

        def mul_body(rr, carry2):
            rb = rr * 8
            for u in range(8):
                msgv[rb + u, :] = msgv[rb + u, :] * ewv[rb + u, :]
            return carry2

        lax.fori_loop(0, CHUNK // 8, mul_body, 0)
        for j in range(NSTREAM):
            pltpu.sync_copy(msgv.at[pl.ds(j * STREAM, STREAM)],
                            acc.at[colv.at[j]], add=True)
        return carry

    lax.fori_loop(0, NCHUNK, chunk_body, 0)

    plsc.subcore_barrier()
    pltpu.sync_copy(acc.at[pl.ds(srow, ROWS_PER_SUBCORE)],
                    out.at[c, pl.ds(srow, ROWS_PER_SUBCORE)])


def _sc_scatter(table, row2, col2, ew16, zrows):
    f = pl.kernel(
        _sc_scatter_body,
        out_type=jax.ShapeDtypeStruct((NUM_SC, NODES_PAD, LANES), _F32),
        mesh=_sc_mesh(),
        scratch_types=[
            pltpu.VMEM((NSTREAM, STREAM), jnp.int32),
            pltpu.VMEM((NSTREAM, STREAM), jnp.int32),
            pltpu.VMEM((CHUNK, LANES), _F32),
            pltpu.VMEM((CHUNK, LANES), _F32),
            pltpu.VMEM_SHARED((NODES_PAD, LANES), _F32),
            pltpu.SemaphoreType.DMA,
        ],
        compiler_params=_SC_PARAMS,
    )
    return f(table, row2, col2, ew16, zrows)


def _sc_gather_body(q2, row2, col2, ef, idxv, buf, sem):
    c = lax.axis_index("c")
    s = lax.axis_index("s")
    tile = s * NUM_SC + c

    def one_side(idx2, lane0, base, r0):
        pltpu.sync_copy(idx2.at[pl.ds(r0, NSTREAM)], idxv)
        cps = [pltpu.async_copy(
                   q2.at[idxv.at[j]],
                   buf.at[pl.ds(j * STREAM, STREAM)], sem)
               for j in range(NSTREAM)]
        for cp in cps:
            cp.wait()
        pltpu.sync_copy(buf, ef.at[pl.ds(base, CHUNK), pl.ds(lane0, CH)])

    def chunk_body(i, carry):
        base = pl.multiple_of(tile * PER_TILE + i * CHUNK, CHUNK)
        r0 = pl.multiple_of(base // STREAM, NSTREAM)
        one_side(row2, 0, base, r0)
        one_side(col2, CH, base, r0)
        return carry

    lax.fori_loop(0, NCHUNK, chunk_body, 0)


def _sc_gather(q2, row2, col2):
    f = pl.kernel(
        _sc_gather_body,
        out_type=jax.ShapeDtypeStruct((EDGES_PAD, 2 * CH), _F32),
        mesh=_sc_mesh(),
        scratch_types=[
            pltpu.VMEM((NSTREAM, STREAM), jnp.int32),
            pltpu.VMEM((CHUNK, CH), _F32),
            pltpu.SemaphoreType.DMA,
        ],
        compiler_params=_SC_PARAMS,
    )
    return f(q2, row2, col2)


def _g1_body(p1_ref, x2_ref, x3_ref, g1_ref):
    a = p1_ref[0] + p1_ref[1]
    g1_ref[...] = a + x2_ref[...] + a[:, 12:13] * x3_ref[...]


def _build_g1(part1, xpad2, xpad3):
    r = ROWS_PER_SUBCORE
    return pl.pallas_call(
        _g1_body,
        grid=(NODES_PAD // r,),
        in_specs=[
            pl.BlockSpec((NUM_SC, r, LANES), lambda i: (0, i, 0)),
            pl.BlockSpec((r, LANES), lambda i: (i, 0)),
            pl.BlockSpec((r, LANES), lambda i: (i, 0)),
        ],
        out_specs=pl.BlockSpec((r, LANES), lambda i: (i, 0)),
        out_shape=jax.ShapeDtypeStruct((NODES_PAD, LANES), _F32),
    )(part1, xpad2, xpad3)


def _q2_body(p2_ref, g1_ref, wg_ref, wh_ref, wk_ref, q2_ref):
    g = p2_ref[0] + p2_ref[1]
    g1 = g1_ref[...]
    q2_ref[...] = (
        jnp.dot(g, wg_ref[...], preferred_element_type=_F32)
        + jnp.dot(g1, wh_ref[...], preferred_element_type=_F32)
        + jnp.dot(g1 * g1[:, 12:13], wk_ref[...], preferred_element_type=_F32))


def _compute_q2(part2, g1, wg, wh, wk):
    r = ROWS_PER_SUBCORE
    wspec = pl.BlockSpec((LANES, CH), lambda i: (0, 0))
    return pl.pallas_call(
        _q2_body,
        grid=(NODES_PAD // r,),
        in_specs=[
            pl.BlockSpec((NUM_SC, r, LANES), lambda i: (0, i, 0)),
            pl.BlockSpec((r, LANES), lambda i: (i, 0)),
            wspec, wspec, wspec,
        ],
        out_specs=pl.BlockSpec((r, CH), lambda i: (i, 0)),
        out_shape=jax.ShapeDtypeStruct((NODES_PAD, CH), _F32),
    )(part2, g1, wg, wh, wk)


def _head_body(ef_ref, w1_ref, be1_ref, w2_ref, be2_ref, out_ref):
    h = jnp.dot(ef_ref[...], w1_ref[...], preferred_element_type=_F32)
    h = jnp.maximum(h + be1_ref[...], 0.0)
    out_ref[...] = jnp.dot(h, w2_ref[...], preferred_element_type=_F32) + be2_ref[...]


def _head(ef, w1, be1, w2, be2, n_edges):
    b = 8000
    return pl.pallas_call(
        _head_body,
        grid=(n_edges // b,),
        in_specs=[
            pl.BlockSpec((b, 2 * CH), lambda i: (i, 0)),
            pl.BlockSpec((2 * CH, 4 * CH), lambda i: (0, 0)),
            pl.BlockSpec((1, 4 * CH), lambda i: (0, 0)),
            pl.BlockSpec((4 * CH, 4), lambda i: (0, 0)),
            pl.BlockSpec((1, 4), lambda i: (0, 0)),
        ],
        out_specs=pl.BlockSpec((b, 4), lambda i: (i, 0)),
        out_shape=jax.ShapeDtypeStruct((n_edges, 4), _F32),
    )(ef, w1, be1, w2, be2)


def kernel(x, edge_index, edge_attr, w1a, b1a, w2a, w3a, b3a,
           w1b, b1b, w2b, w3b, b3b, we1, be1, we2, be2):
    row = edge_index[0]
    col = edge_index[1]
    ew = edge_attr[:, 0]

    pad = EDGES_PAD - N_EDGES
    pad_idx = (jnp.arange(pad, dtype=jnp.int32) * 97) % N_NODES
    rowp = jnp.concatenate([row, pad_idx]).reshape(-1, STREAM)
    colp = jnp.concatenate([col, pad_idx]).reshape(-1, STREAM)
    ewp = jnp.concatenate([ew, jnp.zeros((pad,), _F32)])
    ew16 = jnp.broadcast_to(ewp[:, None], (EDGES_PAD, LANES))

    xp = jnp.pad(x, ((0, NODES_PAD - N_NODES), (0, 0)))
    z4 = jnp.zeros((NODES_PAD, 4), _F32)
    one1 = jnp.ones((NODES_PAD, 1), _F32)
    xpad = jnp.concatenate(
        [xp, jnp.zeros((NODES_PAD, 8), _F32), one1, jnp.zeros((NODES_PAD, 3), _F32)], axis=1)
    xpad2 = jnp.concatenate(
        [z4, xp, jnp.zeros((NODES_PAD, 5), _F32), one1, jnp.zeros((NODES_PAD, 2), _F32)], axis=1)
    xpad3 = jnp.concatenate([jnp.zeros((NODES_PAD, 8), _F32), xp, z4], axis=1)
    zrows = jnp.zeros((ROWS_PER_SUBCORE, LANES), _F32)

    part1 = _sc_scatter(xpad, rowp, colp, ew16, zrows)
    g1 = _build_g1(part1, xpad2, xpad3)
    part2 = _sc_scatter(g1, rowp, colp, ew16, zrows)

    wq = jnp.concatenate(
        [w1a, w3a, -w2a, b1a[None], b3a[None], jnp.zeros((2, CH), _F32)], axis=0)
    wg = wq @ w1b
    wh = (wq @ w3b).at[12].add(b1b).at[13].add(b3b)
    wk = -(wq @ w2b)

    q2 = _compute_q2(part2, g1, wg, wh, wk)
    ef = _sc_gather(q2, rowp, colp)
    return _head(ef, we1, be1[None], we2, be2[None], N_EDGES)


# speedup vs baseline: 8.8535x; 1.3172x over previous
import functools

import jax
import jax.numpy as jnp
from jax import lax
from jax.experimental import pallas as pl
from jax.experimental.pallas import tpu as pltpu
from jax.experimental.pallas import tpu_sc as plsc

N_NODES = 50000
N_EDGES = 800000
CH = 64
LANES = 16
NUM_SC = 2
NUM_SUBCORES = 16
NUM_TILES = NUM_SC * NUM_SUBCORES
CHUNK = 1024
STREAM = 128
NSTREAM = CHUNK // STREAM
NCHUNK = 25
PER_TILE = NCHUNK * CHUNK
EDGES_PAD = NUM_TILES * PER_TILE
EW_ROWS = EDGES_PAD * LANES // 128
NODES_PAD = 50048
ROWS_PER_SUBCORE = NODES_PAD // NUM_SUBCORES

_F32 = jnp.float32


def _sc_mesh():
    return plsc.VectorSubcoreMesh(core_axis_name="c", subcore_axis_name="s")


_SC_PARAMS = pltpu.CompilerParams(use_tc_tiling_on_sc=False)


def _sc_scatter_body(table, row2, col2, ew16, zrows, out,
                     rowv, colv, ewv, msgv, acc, sem):
    c = lax.axis_index("c")
    s = lax.axis_index("s")
    tile = s * NUM_SC + c

    srow = pl.multiple_of(s * ROWS_PER_SUBCORE, 8)
    pltpu.sync_copy(zrows, acc.at[pl.ds(srow, ROWS_PER_SUBCORE)])
    plsc.subcore_barrier()

    def chunk_body(i, carry):
        base = pl.multiple_of(tile * PER_TILE + i * CHUNK, CHUNK)
        r0 = pl.multiple_of(base // STREAM, NSTREAM)
        e0 = pl.multiple_of(base // 8, CHUNK // 8)
        pltpu.sync_copy(row2.at[pl.ds(r0, NSTREAM)], rowv)
        pltpu.sync_copy(col2.at[pl.ds(r0, NSTREAM)], colv)
        pltpu.sync_copy(ew16.at[pl.ds(e0, CHUNK // 8)], ewv)
        cps = [pltpu.async_copy(table.at[rowv.at[j]],
                                msgv.at[pl.ds(j * STREAM, STREAM)], sem)
               for j in range(NSTREAM)]
        for cp in cps:
            cp.wait()

        def mul_body(rr, carry2):
            rb = rr * 8
            for u in range(8):
                msgv[rb + u, :] = msgv[rb + u, :] * ewv[rr, pl.ds(16 * u, 16)]
            return carry2

        lax.fori_loop(0, CHUNK // 8, mul_body, 0)
        for j in range(NSTREAM):
            pltpu.sync_copy(msgv.at[pl.ds(j * STREAM, STREAM)],
                            acc.at[colv.at[j]], add=True)
        return carry

    lax.fori_loop(0, NCHUNK, chunk_body, 0)

    plsc.subcore_barrier()
    pltpu.sync_copy(acc.at[pl.ds(srow, ROWS_PER_SUBCORE)],
                    out.at[c, pl.ds(srow, ROWS_PER_SUBCORE)])


def _sc_scatter(table, row2, col2, ew16, zrows):
    f = pl.kernel(
        _sc_scatter_body,
        out_type=jax.ShapeDtypeStruct((NUM_SC, NODES_PAD, LANES), _F32),
        mesh=_sc_mesh(),
        scratch_types=[
            pltpu.VMEM((NSTREAM, STREAM), jnp.int32),
            pltpu.VMEM((NSTREAM, STREAM), jnp.int32),
            pltpu.VMEM((CHUNK // 8, 128), _F32),
            pltpu.VMEM((CHUNK, LANES), _F32),
            pltpu.VMEM_SHARED((NODES_PAD, LANES), _F32),
            pltpu.SemaphoreType.DMA,
        ],
        compiler_params=_SC_PARAMS,
    )
    return f(table, row2, col2, ew16, zrows)


def _sc_gather_body(q2, row2, col2, ef, idxv, buf, sem):
    c = lax.axis_index("c")
    s = lax.axis_index("s")
    tile = s * NUM_SC + c

    def one_side(idx2, lane0, base, r0):
        pltpu.sync_copy(idx2.at[pl.ds(r0, NSTREAM)], idxv)
        cps = [pltpu.async_copy(
                   q2.at[idxv.at[j]],
                   buf.at[pl.ds(j * STREAM, STREAM)], sem)
               for j in range(NSTREAM)]
        for cp in cps:
            cp.wait()
        pltpu.sync_copy(buf, ef.at[pl.ds(base, CHUNK), pl.ds(lane0, CH)])

    def chunk_body(i, carry):
        base = pl.multiple_of(tile * PER_TILE + i * CHUNK, CHUNK)
        r0 = pl.multiple_of(base // STREAM, NSTREAM)
        one_side(row2, 0, base, r0)
        one_side(col2, CH, base, r0)
        return carry

    lax.fori_loop(0, NCHUNK, chunk_body, 0)


def _sc_gather(q2, row2, col2):
    f = pl.kernel(
        _sc_gather_body,
        out_type=jax.ShapeDtypeStruct((EDGES_PAD, 2 * CH), _F32),
        mesh=_sc_mesh(),
        scratch_types=[
            pltpu.VMEM((NSTREAM, STREAM), jnp.int32),
            pltpu.VMEM((CHUNK, CH), _F32),
            pltpu.SemaphoreType.DMA,
        ],
        compiler_params=_SC_PARAMS,
    )
    return f(q2, row2, col2)


def _g1_body(p1_ref, x2_ref, x3_ref, g1_ref):
    a = p1_ref[0] + p1_ref[1]
    g1_ref[...] = a + x2_ref[...] + a[:, 12:13] * x3_ref[...]


def _build_g1(part1, xpad2, xpad3):
    r = ROWS_PER_SUBCORE
    return pl.pallas_call(
        _g1_body,
        grid=(NODES_PAD // r,),
        in_specs=[
            pl.BlockSpec((NUM_SC, r, LANES), lambda i: (0, i, 0)),
            pl.BlockSpec((r, LANES), lambda i: (i, 0)),
            pl.BlockSpec((r, LANES), lambda i: (i, 0)),
        ],
        out_specs=pl.BlockSpec((r, LANES), lambda i: (i, 0)),
        out_shape=jax.ShapeDtypeStruct((NODES_PAD, LANES), _F32),
    )(part1, xpad2, xpad3)


def _q2_body(p2_ref, g1_ref, wg_ref, wh_ref, wk_ref, q2_ref):
    g = p2_ref[0] + p2_ref[1]
    g1 = g1_ref[...]
    q2_ref[...] = (
        jnp.dot(g, wg_ref[...], preferred_element_type=_F32)
        + jnp.dot(g1, wh_ref[...], preferred_element_type=_F32)
        + jnp.dot(g1 * g1[:, 12:13], wk_ref[...], preferred_element_type=_F32))


def _compute_q2(part2, g1, wg, wh, wk):
    r = ROWS_PER_SUBCORE
    wspec = pl.BlockSpec((LANES, CH), lambda i: (0, 0))
    return pl.pallas_call(
        _q2_body,
        grid=(NODES_PAD // r,),
        in_specs=[
            pl.BlockSpec((NUM_SC, r, LANES), lambda i: (0, i, 0)),
            pl.BlockSpec((r, LANES), lambda i: (i, 0)),
            wspec, wspec, wspec,
        ],
        out_specs=pl.BlockSpec((r, CH), lambda i: (i, 0)),
        out_shape=jax.ShapeDtypeStruct((NODES_PAD, CH), _F32),
    )(part2, g1, wg, wh, wk)


def _head_body(ef_ref, w1_ref, be1_ref, w2_ref, be2_ref, out_ref):
    h = jnp.dot(ef_ref[...], w1_ref[...], preferred_element_type=_F32)
    h = jnp.maximum(h + be1_ref[...], 0.0)
    out_ref[...] = jnp.dot(h, w2_ref[...], preferred_element_type=_F32) + be2_ref[...]


def _head(ef, w1, be1, w2, be2, n_edges):
    b = 8000
    return pl.pallas_call(
        _head_body,
        grid=(n_edges // b,),
        in_specs=[
            pl.BlockSpec((b, 2 * CH), lambda i: (i, 0)),
            pl.BlockSpec((2 * CH, 4 * CH), lambda i: (0, 0)),
            pl.BlockSpec((1, 4 * CH), lambda i: (0, 0)),
            pl.BlockSpec((4 * CH, 4), lambda i: (0, 0)),
            pl.BlockSpec((1, 4), lambda i: (0, 0)),
        ],
        out_specs=pl.BlockSpec((b, 4), lambda i: (i, 0)),
        out_shape=jax.ShapeDtypeStruct((n_edges, 4), _F32),
    )(ef, w1, be1, w2, be2)


def kernel(x, edge_index, edge_attr, w1a, b1a, w2a, w3a, b3a,
           w1b, b1b, w2b, w3b, b3b, we1, be1, we2, be2):
    row = edge_index[0]
    col = edge_index[1]
    ew = edge_attr[:, 0]

    pad = EDGES_PAD - N_EDGES
    pad_idx = (jnp.arange(pad, dtype=jnp.int32) * 97) % N_NODES
    rowp = jnp.concatenate([row, pad_idx]).reshape(-1, STREAM)
    colp = jnp.concatenate([col, pad_idx]).reshape(-1, STREAM)
    ewp = jnp.concatenate([ew, jnp.zeros((pad,), _F32)])
    ew16 = jnp.broadcast_to(ewp[:, None], (EDGES_PAD, LANES)).reshape(EW_ROWS, 128)

    xp = jnp.pad(x, ((0, NODES_PAD - N_NODES), (0, 0)))
    z4 = jnp.zeros((NODES_PAD, 4), _F32)
    one1 = jnp.ones((NODES_PAD, 1), _F32)
    xpad = jnp.concatenate(
        [xp, jnp.zeros((NODES_PAD, 8), _F32), one1, jnp.zeros((NODES_PAD, 3), _F32)], axis=1)
    xpad2 = jnp.concatenate(
        [z4, xp, jnp.zeros((NODES_PAD, 5), _F32), one1, jnp.zeros((NODES_PAD, 2), _F32)], axis=1)
    xpad3 = jnp.concatenate([jnp.zeros((NODES_PAD, 8), _F32), xp, z4], axis=1)
    zrows = jnp.zeros((ROWS_PER_SUBCORE, LANES), _F32)

    part1 = _sc_scatter(xpad, rowp, colp, ew16, zrows)
    g1 = _build_g1(part1, xpad2, xpad3)
    part2 = _sc_scatter(g1, rowp, colp, ew16, zrows)

    wq = jnp.concatenate(
        [w1a, w3a, -w2a, b1a[None], b3a[None], jnp.zeros((2, CH), _F32)], axis=0)
    wg = wq @ w1b
    wh = (wq @ w3b).at[12].add(b1b).at[13].add(b3b)
    wk = -(wq @ w2b)

    q2 = _compute_q2(part2, g1, wg, wh, wk)
    ef = _sc_gather(q2, rowp, colp)
    return _head(ef, we1, be1[None], we2, be2[None], N_EDGES)
